# baseline (speedup 1.0000x reference)
import functools

import jax
import jax.numpy as jnp
from jax import lax
from jax.experimental import pallas as pl
from jax.experimental.pallas import tpu as pltpu
from jax.experimental.pallas import tpu_sc as plsc

N = 100000
E = 1600000
NP = 100352
SENT = N
CH = 128
EROWS = 12800
EPAD = EROWS * CH - E
NC, NS = 2, 16
NW = NC * NS
SPAN = NP // NS
BR = NP // 8


def _sc_mesh():
    return plsc.VectorSubcoreMesh(core_axis_name="c", subcore_axis_name="s")


def _zero_vmem(ref, nrows):
    z = jnp.zeros((16,), jnp.float32)

    def body(i, _):
        ref[i, :] = z
        return 0

    lax.fori_loop(0, nrows, body, 0)


def _zero_vmem1(ref, n):
    z = jnp.zeros((16,), jnp.float32)

    def body(i, _):
        ref[pl.ds(i * 16, 16)] = z
        return 0

    lax.fori_loop(0, n // 16, body, 0)


G1 = 16
G16 = 4


def _degrees(src2, dst2):
    @functools.partial(
        pl.kernel,
        out_type=jax.ShapeDtypeStruct((NC, NP), jnp.float32),
        mesh=_sc_mesh(),
        compiler_params=pltpu.CompilerParams(use_tc_tiling_on_sc=False),
        scratch_types=[
            pltpu.VMEM((G1, CH), jnp.int32),
            pltpu.VMEM((CH,), jnp.float32),
            pltpu.VMEM((SPAN,), jnp.float32),
            pltpu.VMEM_SHARED((NP,), jnp.float32),
            pltpu.SemaphoreType.DMA,
        ],
    )
    def body(src_h, dst_h, out_h, i0_v, ones_v, zb_v, acc_sh, ss0):
        cid = lax.axis_index("c")
        sid = lax.axis_index("s")
        one = jnp.ones((16,), jnp.float32)
        for i in range(CH // 16):
            ones_v[pl.ds(i * 16, 16)] = one
        _zero_vmem1(zb_v, SPAN)
        pltpu.sync_copy(zb_v, acc_sh.at[pl.ds(sid * SPAN, SPAN)])
        plsc.subcore_barrier()

        rows = EROWS // NS
        ng = rows // G1
        base = sid * rows

        def step(i, _):
            r = base + i * G1

            @pl.when(cid == 0)
            def _():
                pltpu.sync_copy(src_h.at[pl.ds(r, G1)], i0_v)

            @pl.when(cid == 1)
            def _():
                pltpu.sync_copy(dst_h.at[pl.ds(r, G1)], i0_v)

            for j in range(G1):
                pltpu.async_copy(ones_v, acc_sh.at[i0_v.at[j]], ss0,
                                 add=True)
            for j in range(G1):
                pltpu.make_async_copy(ones_v, acc_sh.at[i0_v.at[j]],
                                      ss0).wait()
            return 0

        lax.fori_loop(0, ng, step, 0)
        plsc.subcore_barrier()
        pltpu.sync_copy(acc_sh.at[pl.ds(sid * SPAN, SPAN)],
                        out_h.at[cid, pl.ds(sid * SPAN, SPAN)])

    return body(src2, dst2)


def _prop1(g, src2, dst2):
    @functools.partial(
        pl.kernel,
        out_type=jax.ShapeDtypeStruct((NC, NP), jnp.float32),
        mesh=_sc_mesh(),
        compiler_params=pltpu.CompilerParams(use_tc_tiling_on_sc=False),
        scratch_types=[
            pltpu.VMEM((G1, CH), jnp.int32),
            pltpu.VMEM((G1, CH), jnp.int32),
            pltpu.VMEM((G1 * CH,), jnp.float32),
            pltpu.VMEM((SPAN,), jnp.float32),
            pltpu.VMEM_SHARED((NP,), jnp.float32),
            pltpu.SemaphoreType.DMA,
            pltpu.SemaphoreType.DMA,
        ],
    )
    def body(g_h, src_h, dst_h, out_h, s0_v, d0_v, g0_v, zb_v, acc_sh,
             sg0, ss0):
        cid = lax.axis_index("c")
        sid = lax.axis_index("s")
        _zero_vmem1(zb_v, SPAN)
        pltpu.sync_copy(zb_v, acc_sh.at[pl.ds(sid * SPAN, SPAN)])
        plsc.subcore_barrier()

        wid = sid * NC + cid
        rows = EROWS // NW
        ng = rows // G1
        base = wid * rows

        def step(i, _):
            r = base + i * G1
            pltpu.sync_copy(src_h.at[pl.ds(r, G1)], s0_v)
            pltpu.sync_copy(dst_h.at[pl.ds(r, G1)], d0_v)
            for j in range(G1):
                pltpu.async_copy(g_h.at[s0_v.at[j]],
                                 g0_v.at[pl.ds(j * CH, CH)], sg0)
            for j in range(G1):
                pltpu.make_async_copy(g_h.at[s0_v.at[j]],
                                      g0_v.at[pl.ds(j * CH, CH)],
                                      sg0).wait()
            for j in range(G1):
                pltpu.async_copy(g0_v.at[pl.ds(j * CH, CH)],
                                 acc_sh.at[d0_v.at[j]], ss0, add=True)
            for j in range(G1):
                pltpu.make_async_copy(g0_v.at[pl.ds(j * CH, CH)],
                                      acc_sh.at[d0_v.at[j]], ss0).wait()
            return 0

        lax.fori_loop(0, ng, step, 0)
        plsc.subcore_barrier()
        pltpu.sync_copy(acc_sh.at[pl.ds(sid * SPAN, SPAN)],
                        out_h.at[cid, pl.ds(sid * SPAN, SPAN)])

    return body(g, src2, dst2)


def _prop16_scratch():
    return [
        pltpu.VMEM((G16, CH), jnp.int32),
        pltpu.VMEM((G16, CH), jnp.int32),
        pltpu.VMEM((G16, CH), jnp.int32),
        pltpu.VMEM((G16, CH), jnp.int32),
        pltpu.VMEM((G16 * CH, 16), jnp.float32),
        pltpu.VMEM((G16 * CH, 16), jnp.float32),
        pltpu.VMEM_SHARED((NP, 16), jnp.float32),
        pltpu.SemaphoreType.DMA,
        pltpu.SemaphoreType.DMA,
        pltpu.SemaphoreType.DMA,
        pltpu.SemaphoreType.DMA,
    ]


def _zero_acc16(gb_v, acc_sh, sid):
    _zero_vmem(gb_v, G16 * CH)
    blk = G16 * CH
    nfull, rem = SPAN // blk, SPAN % blk
    for t in range(nfull):
        pltpu.sync_copy(gb_v, acc_sh.at[pl.ds(sid * SPAN + t * blk, blk)])
    if rem:
        pltpu.sync_copy(gb_v.at[pl.ds(0, rem)],
                        acc_sh.at[pl.ds(sid * SPAN + nfull * blk, rem)])


def _prop16_body(g_ref_fn, base_fn, ng):
    def body(g_h, src_h, dst_h, out_h, s0_v, s1_v, d0_v, d1_v, g0_v, g1_v,
             acc_sh, sg0, sg1, ss0, ss1):
        cid = lax.axis_index("c")
        sid = lax.axis_index("s")
        _zero_acc16(g0_v, acc_sh, sid)
        plsc.subcore_barrier()

        base = base_fn(cid, sid)
        gt = g_ref_fn(g_h, cid)

        def fire_g(sv, gv, sem):
            for j in range(G16):
                pltpu.async_copy(gt.at[sv.at[j]],
                                 gv.at[pl.ds(j * CH, CH)], sem)

        def wait_g(sv, gv, sem):
            for j in range(G16):
                pltpu.make_async_copy(gt.at[sv.at[j]],
                                      gv.at[pl.ds(j * CH, CH)], sem).wait()

        def fire_s(dv, gv, sem):
            for j in range(G16):
                pltpu.async_copy(gv.at[pl.ds(j * CH, CH)],
                                 acc_sh.at[dv.at[j]], sem, add=True)

        def wait_s(dv, gv, sem):
            for j in range(G16):
                pltpu.make_async_copy(gv.at[pl.ds(j * CH, CH)],
                                      acc_sh.at[dv.at[j]], sem).wait()

        def step(i, _):
            ra = base + (2 * i) * G16
            rb = ra + G16
            pltpu.sync_copy(src_h.at[pl.ds(ra, G16)], s0_v)
            fire_g(s0_v, g0_v, sg0)
            pltpu.sync_copy(dst_h.at[pl.ds(ra, G16)], d0_v)
            pltpu.sync_copy(src_h.at[pl.ds(rb, G16)], s1_v)
            pltpu.sync_copy(dst_h.at[pl.ds(rb, G16)], d1_v)
            wait_g(s0_v, g0_v, sg0)
            fire_s(d0_v, g0_v, ss0)
            fire_g(s1_v, g1_v, sg1)
            wait_g(s1_v, g1_v, sg1)
            fire_s(d1_v, g1_v, ss1)
            wait_s(d0_v, g0_v, ss0)
            wait_s(d1_v, g1_v, ss1)
            return 0

        lax.fori_loop(0, ng // 2, step, 0)
        plsc.subcore_barrier()
        pltpu.sync_copy(acc_sh.at[pl.ds(sid * SPAN, SPAN)],
                        out_h.at[cid, pl.ds(sid * SPAN, SPAN)])

    return body


def _prop16(g, src2, dst2):
    body = _prop16_body(
        g_ref_fn=lambda g_h, cid: g_h,
        base_fn=lambda cid, sid: (sid * NC + cid) * (EROWS // NW),
        ng=(EROWS // NW) // G16,
    )
    k = functools.partial(
        pl.kernel,
        out_type=jax.ShapeDtypeStruct((NC, NP, 16), jnp.float32),
        mesh=_sc_mesh(),
        compiler_params=pltpu.CompilerParams(use_tc_tiling_on_sc=False),
        scratch_types=_prop16_scratch(),
    )(body)
    return k(g, src2, dst2)


def _prop32(g2, src2, dst2):
    body = _prop16_body(
        g_ref_fn=lambda g_h, cid: g_h.at[cid],
        base_fn=lambda cid, sid: sid * (EROWS // NS),
        ng=(EROWS // NS) // G16,
    )
    k = functools.partial(
        pl.kernel,
        out_type=jax.ShapeDtypeStruct((NC, NP, 16), jnp.float32),
        mesh=_sc_mesh(),
        compiler_params=pltpu.CompilerParams(use_tc_tiling_on_sc=False),
        scratch_types=_prop16_scratch(),
    )(body)
    return k(g2, src2, dst2)


BR2 = NP // 32


def _col2(d=1):
    return pl.BlockSpec((BR2, d), lambda i: (i, 0))


def _feat2(d):
    return pl.BlockSpec((NC, BR2, d), lambda i: (0, i, 0))


def _sq(nd=1):
    if nd == 1:
        return pl.BlockSpec((784, 128), lambda i: (0, 0))
    return pl.BlockSpec((NC, 784, 128), lambda i: (0, 0, 0))


def _full(shape):
    return pl.BlockSpec(shape, lambda i: tuple(0 for _ in shape))


def _tc_prep(deg_out, deg_in, xp):
    def body(do_r, di_r, x_r, ns_r, nd_r, g1_r):
        do = do_r[...]
        di = di_r[...]
        ns = lax.rsqrt(jnp.where(do > 0, do, 1.0))
        nd = lax.rsqrt(jnp.where(di > 0, di, 1.0))
        ns_r[...] = ns
        nd_r[...] = nd
        g1_r[...] = ns * x_r[...]

    out = jax.ShapeDtypeStruct((784, 128), jnp.float32)
    return pl.pallas_call(
        body, grid=(1,),
        in_specs=[_sq(), _sq(), _sq()],
        out_specs=[_sq(), _sq(), _sq()],
        out_shape=[out, out, out],
    )(deg_out, deg_in, xp)


def _tc_l1(s1, ns, nd, W1, b1):
    def body(s_r, ns_r, nd_r, w_r, b_r, o_r):
        p = nd_r[...] * (s_r[0] + s_r[1])
        h = jnp.maximum(p * w_r[...] + b_r[...], 0.0)
        o_r[...] = ns_r[...] * h

    return pl.pallas_call(
        body, grid=(32,),
        in_specs=[_feat2(1), _col2(), _col2(), _full((1, 16)),
                  _full((1, 16))],
        out_specs=_col2(16),
        out_shape=jax.ShapeDtypeStruct((NP, 16), jnp.float32),
    )(s1, ns, nd, W1, b1)


def _tc_l2(s2, ns, nd, W2, b2):
    def body(s_r, ns_r, nd_r, w_r, b_r, o_r):
        p = nd_r[...] * (s_r[0] + s_r[1])
        h = jnp.maximum(
            jnp.dot(p, w_r[...], preferred_element_type=jnp.float32)
            + b_r[...], 0.0)
        g = ns_r[...] * h
        o_r[0] = g[:, :16]
        o_r[1] = g[:, 16:]

    return pl.pallas_call(
        body, grid=(32,),
        in_specs=[_feat2(16), _col2(), _col2(), _full((16, 32)),
                  _full((1, 32))],
        out_specs=_feat2(16),
        out_shape=jax.ShapeDtypeStruct((NC, NP, 16), jnp.float32),
    )(s2, ns, nd, W2, b2)


def _tc_l3(s3, ns, nd, W3, b3, W4):
    def body(s_r, ns_r, nd_r, w3_r, b3_r, w4_r, o_r):
        p = nd_r[...] * jnp.concatenate([s_r[0], s_r[1]], axis=1)
        h = jnp.maximum(
            jnp.dot(p, w3_r[...], preferred_element_type=jnp.float32)
            + b3_r[...], 0.0)
        t = jnp.dot(h, w4_r[...], preferred_element_type=jnp.float32)
        g = ns_r[...] * t
        o_r[0] = g[:, :16]
        o_r[1] = g[:, 16:]

    return pl.pallas_call(
        body, grid=(32,),
        in_specs=[_feat2(16), _col2(), _col2(), _full((32, 64)),
                  _full((1, 64)), _full((64, 32))],
        out_specs=_feat2(16),
        out_shape=jax.ShapeDtypeStruct((NC, NP, 16), jnp.float32),
    )(s3, ns, nd, W3, b3, W4)


def _tc_l4(s4, ns, nd, b4, W5):
    def body(s_r, ns_r, nd_r, b4_r, w5_r, o_r):
        p = nd_r[...] * jnp.concatenate([s_r[0], s_r[1]], axis=1)
        h = jnp.maximum(p + b4_r[...], 0.0)
        t = jnp.dot(h, w5_r[...], preferred_element_type=jnp.float32)
        o_r[...] = ns_r[...] * t

    return pl.pallas_call(
        body, grid=(32,),
        in_specs=[_feat2(16), _col2(), _col2(), _full((1, 32)),
                  _full((32, 1))],
        out_specs=_col2(),
        out_shape=jax.ShapeDtypeStruct((NP, 1), jnp.float32),
    )(s4, ns, nd, b4, W5)


def _tc_out(s5, nd, b5):
    def body(s_r, nd_r, b_r, o_r):
        p = nd_r[...] * (s_r[0] + s_r[1]) + b_r[...]
        o_r[...] = 1.0 / (1.0 + jnp.exp(-p))

    return pl.pallas_call(
        body, grid=(1,),
        in_specs=[_sq(2), _sq(), _full((1, 1))],
        out_specs=_sq(),
        out_shape=jax.ShapeDtypeStruct((784, 128), jnp.float32),
    )(s5, nd, b5)


def kernel(x, edge_index, W1, b1, W2, b2, W3, b3, W4, b4, W5, b5):
    tr = SENT + jnp.arange(EPAD, dtype=jnp.int32) % (NP - N)
    pad = jnp.stack([tr, tr])
    ei = jnp.concatenate([edge_index.astype(jnp.int32), pad], axis=1)
    src2 = ei[0].reshape(EROWS, CH)
    dst2 = ei[1].reshape(EROWS, CH)
    xp = jnp.pad(x.reshape(-1), (0, NP - N)).reshape(784, 128)

    deg = _degrees(src2, dst2)
    ns2, nd2, g12 = _tc_prep(deg[0].reshape(784, 128),
                             deg[1].reshape(784, 128), xp)
    ns = ns2.reshape(NP, 1)
    nd = nd2.reshape(NP, 1)

    s1 = _prop1(g12.reshape(NP), src2, dst2)
    g2 = _tc_l1(s1.reshape(NC, NP, 1), ns, nd, W1, b1.reshape(1, 16))

    s2 = _prop16(g2, src2, dst2)
    g3 = _tc_l2(s2, ns, nd, W2, b2.reshape(1, 32))

    s3 = _prop32(g3, src2, dst2)
    g4 = _tc_l3(s3, ns, nd, W3, b3.reshape(1, 64), W4)

    s4 = _prop32(g4, src2, dst2)
    g5 = _tc_l4(s4, ns, nd, b4.reshape(1, 32), W5)

    s5 = _prop1(g5.reshape(NP), src2, dst2)
    out = _tc_out(s5.reshape(NC, 784, 128), nd2, b5.reshape(1, 1))

    return out.reshape(NP)[:N].reshape(1, N)


# speedup vs baseline: 1.0162x; 1.0162x over previous
import functools

import jax
import jax.numpy as jnp
from jax import lax
from jax.experimental import pallas as pl
from jax.experimental.pallas import tpu as pltpu
from jax.experimental.pallas import tpu_sc as plsc

N = 100000
E = 1600000
NP = 100352
SENT = N
CH = 128
EROWS = 12800
EPAD = EROWS * CH - E
NC, NS = 2, 16
NW = NC * NS
SPAN = NP // NS
BR = NP // 8


def _sc_mesh():
    return plsc.VectorSubcoreMesh(core_axis_name="c", subcore_axis_name="s")


def _zero_vmem(ref, nrows):
    z = jnp.zeros((16,), jnp.float32)

    def body(i, _):
        ref[i, :] = z
        return 0

    lax.fori_loop(0, nrows, body, 0)


def _zero_vmem1(ref, n):
    z = jnp.zeros((16,), jnp.float32)

    def body(i, _):
        ref[pl.ds(i * 16, 16)] = z
        return 0

    lax.fori_loop(0, n // 16, body, 0)


G1 = 16
G16 = 8


def _degrees(src2, dst2):
    @functools.partial(
        pl.kernel,
        out_type=jax.ShapeDtypeStruct((NC, NP), jnp.float32),
        mesh=_sc_mesh(),
        compiler_params=pltpu.CompilerParams(use_tc_tiling_on_sc=False),
        scratch_types=[
            pltpu.VMEM((G1, CH), jnp.int32),
            pltpu.VMEM((CH,), jnp.float32),
            pltpu.VMEM((SPAN,), jnp.float32),
            pltpu.VMEM_SHARED((NP,), jnp.float32),
            pltpu.SemaphoreType.DMA,
        ],
    )
    def body(src_h, dst_h, out_h, i0_v, ones_v, zb_v, acc_sh, ss0):
        cid = lax.axis_index("c")
        sid = lax.axis_index("s")
        one = jnp.ones((16,), jnp.float32)
        for i in range(CH // 16):
            ones_v[pl.ds(i * 16, 16)] = one
        _zero_vmem1(zb_v, SPAN)
        pltpu.sync_copy(zb_v, acc_sh.at[pl.ds(sid * SPAN, SPAN)])
        plsc.subcore_barrier()

        rows = EROWS // NS
        ng = rows // G1
        base = sid * rows

        def step(i, _):
            r = base + i * G1

            @pl.when(cid == 0)
            def _():
                pltpu.sync_copy(src_h.at[pl.ds(r, G1)], i0_v)

            @pl.when(cid == 1)
            def _():
                pltpu.sync_copy(dst_h.at[pl.ds(r, G1)], i0_v)

            for j in range(G1):
                pltpu.async_copy(ones_v, acc_sh.at[i0_v.at[j]], ss0,
                                 add=True)
            for j in range(G1):
                pltpu.make_async_copy(ones_v, acc_sh.at[i0_v.at[j]],
                                      ss0).wait()
            return 0

        lax.fori_loop(0, ng, step, 0)
        plsc.subcore_barrier()
        pltpu.sync_copy(acc_sh.at[pl.ds(sid * SPAN, SPAN)],
                        out_h.at[cid, pl.ds(sid * SPAN, SPAN)])

    return body(src2, dst2)


def _prop1(g, src2, dst2):
    @functools.partial(
        pl.kernel,
        out_type=jax.ShapeDtypeStruct((NC, NP), jnp.float32),
        mesh=_sc_mesh(),
        compiler_params=pltpu.CompilerParams(use_tc_tiling_on_sc=False),
        scratch_types=[
            pltpu.VMEM((G1, CH), jnp.int32),
            pltpu.VMEM((G1, CH), jnp.int32),
            pltpu.VMEM((G1 * CH,), jnp.float32),
            pltpu.VMEM((SPAN,), jnp.float32),
            pltpu.VMEM_SHARED((NP,), jnp.float32),
            pltpu.SemaphoreType.DMA,
            pltpu.SemaphoreType.DMA,
        ],
    )
    def body(g_h, src_h, dst_h, out_h, s0_v, d0_v, g0_v, zb_v, acc_sh,
             sg0, ss0):
        cid = lax.axis_index("c")
        sid = lax.axis_index("s")
        _zero_vmem1(zb_v, SPAN)
        pltpu.sync_copy(zb_v, acc_sh.at[pl.ds(sid * SPAN, SPAN)])
        plsc.subcore_barrier()

        wid = sid * NC + cid
        rows = EROWS // NW
        ng = rows // G1
        base = wid * rows

        def step(i, _):
            r = base + i * G1
            pltpu.sync_copy(src_h.at[pl.ds(r, G1)], s0_v)
            pltpu.sync_copy(dst_h.at[pl.ds(r, G1)], d0_v)
            for j in range(G1):
                pltpu.async_copy(g_h.at[s0_v.at[j]],
                                 g0_v.at[pl.ds(j * CH, CH)], sg0)
            for j in range(G1):
                pltpu.make_async_copy(g_h.at[s0_v.at[j]],
                                      g0_v.at[pl.ds(j * CH, CH)],
                                      sg0).wait()
            for j in range(G1):
                pltpu.async_copy(g0_v.at[pl.ds(j * CH, CH)],
                                 acc_sh.at[d0_v.at[j]], ss0, add=True)
            for j in range(G1):
                pltpu.make_async_copy(g0_v.at[pl.ds(j * CH, CH)],
                                      acc_sh.at[d0_v.at[j]], ss0).wait()
            return 0

        lax.fori_loop(0, ng, step, 0)
        plsc.subcore_barrier()
        pltpu.sync_copy(acc_sh.at[pl.ds(sid * SPAN, SPAN)],
                        out_h.at[cid, pl.ds(sid * SPAN, SPAN)])

    return body(g, src2, dst2)


def _prop16_scratch():
    return [
        pltpu.VMEM((G16, CH), jnp.int32),
        pltpu.VMEM((G16, CH), jnp.int32),
        pltpu.VMEM((G16 * CH, 16), jnp.float32),
        pltpu.VMEM_SHARED((NP, 16), jnp.float32),
        pltpu.SemaphoreType.DMA,
        pltpu.SemaphoreType.DMA,
    ]


def _zero_acc16(gb_v, acc_sh, sid):
    _zero_vmem(gb_v, G16 * CH)
    blk = G16 * CH
    nfull, rem = SPAN // blk, SPAN % blk
    for t in range(nfull):
        pltpu.sync_copy(gb_v, acc_sh.at[pl.ds(sid * SPAN + t * blk, blk)])
    if rem:
        pltpu.sync_copy(gb_v.at[pl.ds(0, rem)],
                        acc_sh.at[pl.ds(sid * SPAN + nfull * blk, rem)])


def _prop16_body(g_ref_fn, base_fn, ng):
    def body(g_h, src_h, dst_h, out_h, s0_v, d0_v, g0_v, acc_sh, sg0, ss0):
        cid = lax.axis_index("c")
        sid = lax.axis_index("s")
        _zero_acc16(g0_v, acc_sh, sid)
        plsc.subcore_barrier()

        base = base_fn(cid, sid)
        gt = g_ref_fn(g_h, cid)

        def step(i, _):
            r = base + i * G16
            pltpu.sync_copy(src_h.at[pl.ds(r, G16)], s0_v)
            pltpu.sync_copy(dst_h.at[pl.ds(r, G16)], d0_v)
            for j in range(G16):
                pltpu.async_copy(gt.at[s0_v.at[j]],
                                 g0_v.at[pl.ds(j * CH, CH)], sg0)
            for j in range(G16):
                pltpu.make_async_copy(gt.at[s0_v.at[j]],
                                      g0_v.at[pl.ds(j * CH, CH)],
                                      sg0).wait()
            for j in range(G16):
                pltpu.async_copy(g0_v.at[pl.ds(j * CH, CH)],
                                 acc_sh.at[d0_v.at[j]], ss0, add=True)
            for j in range(G16):
                pltpu.make_async_copy(g0_v.at[pl.ds(j * CH, CH)],
                                      acc_sh.at[d0_v.at[j]], ss0).wait()
            return 0

        lax.fori_loop(0, ng, step, 0)
        plsc.subcore_barrier()
        pltpu.sync_copy(acc_sh.at[pl.ds(sid * SPAN, SPAN)],
                        out_h.at[cid, pl.ds(sid * SPAN, SPAN)])

    return body


def _prop16(g, src2, dst2):
    body = _prop16_body(
        g_ref_fn=lambda g_h, cid: g_h,
        base_fn=lambda cid, sid: (sid * NC + cid) * (EROWS // NW),
        ng=(EROWS // NW) // G16,
    )
    k = functools.partial(
        pl.kernel,
        out_type=jax.ShapeDtypeStruct((NC, NP, 16), jnp.float32),
        mesh=_sc_mesh(),
        compiler_params=pltpu.CompilerParams(use_tc_tiling_on_sc=False),
        scratch_types=_prop16_scratch(),
    )(body)
    return k(g, src2, dst2)


def _prop32(g2, src2, dst2):
    body = _prop16_body(
        g_ref_fn=lambda g_h, cid: g_h.at[cid],
        base_fn=lambda cid, sid: sid * (EROWS // NS),
        ng=(EROWS // NS) // G16,
    )
    k = functools.partial(
        pl.kernel,
        out_type=jax.ShapeDtypeStruct((NC, NP, 16), jnp.float32),
        mesh=_sc_mesh(),
        compiler_params=pltpu.CompilerParams(use_tc_tiling_on_sc=False),
        scratch_types=_prop16_scratch(),
    )(body)
    return k(g2, src2, dst2)


BR2 = NP // 32


def _col2(d=1):
    return pl.BlockSpec((BR2, d), lambda i: (i, 0))


def _feat2(d):
    return pl.BlockSpec((NC, BR2, d), lambda i: (0, i, 0))


def _sq(nd=1):
    if nd == 1:
        return pl.BlockSpec((784, 128), lambda i: (0, 0))
    return pl.BlockSpec((NC, 784, 128), lambda i: (0, 0, 0))


def _full(shape):
    return pl.BlockSpec(shape, lambda i: tuple(0 for _ in shape))


def _tc_prep(deg_out, deg_in, xp):
    def body(do_r, di_r, x_r, ns_r, nd_r, g1_r):
        do = do_r[...]
        di = di_r[...]
        ns = lax.rsqrt(jnp.where(do > 0, do, 1.0))
        nd = lax.rsqrt(jnp.where(di > 0, di, 1.0))
        ns_r[...] = ns
        nd_r[...] = nd
        g1_r[...] = ns * x_r[...]

    out = jax.ShapeDtypeStruct((784, 128), jnp.float32)
    return pl.pallas_call(
        body, grid=(1,),
        in_specs=[_sq(), _sq(), _sq()],
        out_specs=[_sq(), _sq(), _sq()],
        out_shape=[out, out, out],
    )(deg_out, deg_in, xp)


def _tc_l1(s1, ns, nd, W1, b1):
    def body(s_r, ns_r, nd_r, w_r, b_r, o_r):
        p = nd_r[...] * (s_r[0] + s_r[1])
        h = jnp.maximum(p * w_r[...] + b_r[...], 0.0)
        o_r[...] = ns_r[...] * h

    return pl.pallas_call(
        body, grid=(32,),
        in_specs=[_feat2(1), _col2(), _col2(), _full((1, 16)),
                  _full((1, 16))],
        out_specs=_col2(16),
        out_shape=jax.ShapeDtypeStruct((NP, 16), jnp.float32),
    )(s1, ns, nd, W1, b1)


def _tc_l2(s2, ns, nd, W2, b2):
    def body(s_r, ns_r, nd_r, w_r, b_r, o_r):
        p = nd_r[...] * (s_r[0] + s_r[1])
        h = jnp.maximum(
            jnp.dot(p, w_r[...], preferred_element_type=jnp.float32)
            + b_r[...], 0.0)
        g = ns_r[...] * h
        o_r[0] = g[:, :16]
        o_r[1] = g[:, 16:]

    return pl.pallas_call(
        body, grid=(32,),
        in_specs=[_feat2(16), _col2(), _col2(), _full((16, 32)),
                  _full((1, 32))],
        out_specs=_feat2(16),
        out_shape=jax.ShapeDtypeStruct((NC, NP, 16), jnp.float32),
    )(s2, ns, nd, W2, b2)


def _tc_l3(s3, ns, nd, W3, b3, W4):
    def body(s_r, ns_r, nd_r, w3_r, b3_r, w4_r, o_r):
        p = nd_r[...] * jnp.concatenate([s_r[0], s_r[1]], axis=1)
        h = jnp.maximum(
            jnp.dot(p, w3_r[...], preferred_element_type=jnp.float32)
            + b3_r[...], 0.0)
        t = jnp.dot(h, w4_r[...], preferred_element_type=jnp.float32)
        g = ns_r[...] * t
        o_r[0] = g[:, :16]
        o_r[1] = g[:, 16:]

    return pl.pallas_call(
        body, grid=(32,),
        in_specs=[_feat2(16), _col2(), _col2(), _full((32, 64)),
                  _full((1, 64)), _full((64, 32))],
        out_specs=_feat2(16),
        out_shape=jax.ShapeDtypeStruct((NC, NP, 16), jnp.float32),
    )(s3, ns, nd, W3, b3, W4)


def _tc_l4(s4, ns, nd, b4, W5):
    def body(s_r, ns_r, nd_r, b4_r, w5_r, o_r):
        p = nd_r[...] * jnp.concatenate([s_r[0], s_r[1]], axis=1)
        h = jnp.maximum(p + b4_r[...], 0.0)
        t = jnp.dot(h, w5_r[...], preferred_element_type=jnp.float32)
        o_r[...] = ns_r[...] * t

    return pl.pallas_call(
        body, grid=(32,),
        in_specs=[_feat2(16), _col2(), _col2(), _full((1, 32)),
                  _full((32, 1))],
        out_specs=_col2(),
        out_shape=jax.ShapeDtypeStruct((NP, 1), jnp.float32),
    )(s4, ns, nd, b4, W5)


def _tc_out(s5, nd, b5):
    def body(s_r, nd_r, b_r, o_r):
        p = nd_r[...] * (s_r[0] + s_r[1]) + b_r[...]
        o_r[...] = 1.0 / (1.0 + jnp.exp(-p))

    return pl.pallas_call(
        body, grid=(1,),
        in_specs=[_sq(2), _sq(), _full((1, 1))],
        out_specs=_sq(),
        out_shape=jax.ShapeDtypeStruct((784, 128), jnp.float32),
    )(s5, nd, b5)


def kernel(x, edge_index, W1, b1, W2, b2, W3, b3, W4, b4, W5, b5):
    tr = SENT + jnp.arange(EPAD, dtype=jnp.int32) % (NP - N)
    pad = jnp.stack([tr, tr])
    ei = jnp.concatenate([edge_index.astype(jnp.int32), pad], axis=1)
    src2 = ei[0].reshape(EROWS, CH)
    dst2 = ei[1].reshape(EROWS, CH)
    xp = jnp.pad(x.reshape(-1), (0, NP - N)).reshape(784, 128)

    deg = _degrees(src2, dst2)
    ns2, nd2, g12 = _tc_prep(deg[0].reshape(784, 128),
                             deg[1].reshape(784, 128), xp)
    ns = ns2.reshape(NP, 1)
    nd = nd2.reshape(NP, 1)

    s1 = _prop1(g12.reshape(NP), src2, dst2)
    g2 = _tc_l1(s1.reshape(NC, NP, 1), ns, nd, W1, b1.reshape(1, 16))

    s2 = _prop16(g2, src2, dst2)
    g3 = _tc_l2(s2, ns, nd, W2, b2.reshape(1, 32))

    s3 = _prop32(g3, src2, dst2)
    g4 = _tc_l3(s3, ns, nd, W3, b3.reshape(1, 64), W4)

    s4 = _prop32(g4, src2, dst2)
    g5 = _tc_l4(s4, ns, nd, b4.reshape(1, 32), W5)

    s5 = _prop1(g5.reshape(NP), src2, dst2)
    out = _tc_out(s5.reshape(NC, 784, 128), nd2, b5.reshape(1, 1))

    return out.reshape(NP)[:N].reshape(1, N)
